# Initial kernel scaffold; baseline (speedup 1.0000x reference)
import jax
import jax.numpy as jnp
from jax.experimental import pallas as pl


def kernel(node_feature, qs, W_rel, W_self, edge_index, edge_type, node_type, graph_ids):
    raise NotImplementedError("write your pallas kernel here")


# speedup vs baseline: 5.9356x; 5.9356x over previous
import functools

import jax
import jax.numpy as jnp
from jax import lax
from jax.experimental import pallas as pl
from jax.experimental.pallas import tpu as pltpu
from jax.experimental.pallas import tpu_sc as plsc

N_NODES = 10000
D_FEAT = 128
EMB = 32
N_REL = 3
N_GRAPHS = 64

NW = 32
EDGE_BLK = 128
E_PAD = 327680
N_EDGE_ROWS = E_PAD // EDGE_BLK
BLKS_PER_W = N_EDGE_ROWS // NW
ACC_ROWS = 10112
ROWS_PER_TILE = ACC_ROWS // 16
MM_GRID = 10
MM_BLK = N_NODES // MM_GRID
EROW_BLK = N_EDGE_ROWS // MM_GRID
RED_GRID = 8
RED_BLK = ACC_ROWS // RED_GRID


def _mm_body(x_ref, w_ref, s_ref, t_ref, y_ref, t2_ref, g_ref):
    r = pl.program_id(0)
    y = jnp.dot(x_ref[...], w_ref[...], preferred_element_type=jnp.float32)
    y_ref[...] = y
    lane = lax.broadcasted_iota(jnp.int32, (MM_BLK, D_FEAT), 1)
    t2_ref[...] = jnp.where((lane // EMB) == r, y, 0.0)
    g_ref[...] = t_ref[...] * N_NODES + s_ref[...]


_mm_call = pl.pallas_call(
    _mm_body,
    grid=(N_REL, MM_GRID),
    in_specs=[
        pl.BlockSpec((MM_BLK, D_FEAT), lambda r, i: (i, 0)),
        pl.BlockSpec((D_FEAT, D_FEAT), lambda r, i: (0, 0)),
        pl.BlockSpec((EROW_BLK, EDGE_BLK), lambda r, i: (i, 0)),
        pl.BlockSpec((EROW_BLK, EDGE_BLK), lambda r, i: (i, 0)),
    ],
    out_specs=[
        pl.BlockSpec((MM_BLK, D_FEAT), lambda r, i: (i, 0)),
        pl.BlockSpec((MM_BLK, D_FEAT), lambda r, i: (r * MM_GRID + i, 0)),
        pl.BlockSpec((EROW_BLK, EDGE_BLK), lambda r, i: (i, 0)),
    ],
    out_shape=[
        jax.ShapeDtypeStruct((N_NODES, D_FEAT), jnp.float32),
        jax.ShapeDtypeStruct((N_REL * N_NODES, D_FEAT), jnp.float32),
        jax.ShapeDtypeStruct((N_EDGE_ROWS, EDGE_BLK), jnp.int32),
    ],
)


@functools.cache
def _edge_agg_call():
    return functools.partial(
        pl.kernel,
        out_type=jax.ShapeDtypeStruct((2, ACC_ROWS, D_FEAT), jnp.float32),
        mesh=plsc.VectorSubcoreMesh(core_axis_name="c", subcore_axis_name="s",
                                    num_cores=2, num_subcores=16),
        scratch_types=[
            pltpu.VMEM((BLKS_PER_W // 2, EDGE_BLK), jnp.int32),
            pltpu.VMEM((BLKS_PER_W // 2, EDGE_BLK), jnp.int32),
            pltpu.VMEM((EDGE_BLK, D_FEAT), jnp.float32),
            pltpu.VMEM((EDGE_BLK, D_FEAT), jnp.float32),
            pltpu.VMEM_SHARED((ACC_ROWS, D_FEAT), jnp.float32),
            pltpu.SemaphoreType.DMA,
            pltpu.SemaphoreType.DMA,
        ],
    )(_edge_agg)


def _edge_agg(t2_hbm, gidx_hbm, dst_hbm, zeros_hbm, out_hbm,
              gidx_v, dst_v, rows_a, rows_b, acc_sh, sem_a, sem_b):
    cid = lax.axis_index("c")
    sid = lax.axis_index("s")
    wid = cid * 16 + sid
    pltpu.sync_copy(zeros_hbm.at[pl.ds(sid * ROWS_PER_TILE, ROWS_PER_TILE)],
                    acc_sh.at[pl.ds(sid * ROWS_PER_TILE, ROWS_PER_TILE)])
    plsc.subcore_barrier()

    def body(i, carry):
        j = i * 2
        cpa = pltpu.async_copy(t2_hbm.at[gidx_v.at[j]], rows_a, sem_a)
        cpb = pltpu.async_copy(t2_hbm.at[gidx_v.at[j + 1]], rows_b, sem_b)
        cpa.wait()
        pltpu.sync_copy(rows_a, acc_sh.at[dst_v.at[j]], add=True)
        cpb.wait()
        pltpu.sync_copy(rows_b, acc_sh.at[dst_v.at[j + 1]], add=True)
        return carry

    half = BLKS_PER_W // 2
    for p in range(2):
        blk0 = wid * BLKS_PER_W + p * half
        pltpu.sync_copy(gidx_hbm.at[pl.ds(blk0, half)], gidx_v)
        pltpu.sync_copy(dst_hbm.at[pl.ds(blk0, half)], dst_v)
        lax.fori_loop(0, half // 2, body, 0)
    plsc.subcore_barrier()
    pltpu.sync_copy(acc_sh.at[pl.ds(sid * ROWS_PER_TILE, ROWS_PER_TILE)],
                    out_hbm.at[cid, pl.ds(sid * ROWS_PER_TILE, ROWS_PER_TILE)])


def _reduce_body(a_ref, o_ref):
    s = a_ref[0] + a_ref[1]
    o_ref[...] = (s[:, 0:EMB] + s[:, EMB:2 * EMB] + s[:, 2 * EMB:3 * EMB])


_reduce_call = pl.pallas_call(
    _reduce_body,
    grid=(RED_GRID,),
    in_specs=[pl.BlockSpec((2, RED_BLK, D_FEAT), lambda i: (0, i, 0))],
    out_specs=pl.BlockSpec((RED_BLK, EMB), lambda i: (i, 0)),
    out_shape=jax.ShapeDtypeStruct((ACC_ROWS, EMB), jnp.float32),
)


def _readout_body(h0_ref, agg_ref, qs_ref, nt_ref, gid_ref, o_ref):
    h = h0_ref[...] + agg_ref[:N_NODES, :]
    w = jnp.maximum(h, 0.0)
    coef = (qs_ref[...] + 1.0) * (nt_ref[...] == 0).astype(jnp.float32)
    gid = gid_ref[...]
    iota = lax.broadcasted_iota(jnp.int32, (N_GRAPHS, N_NODES), 0)
    a = jnp.where(gid == iota, coef, 0.0)
    o_ref[...] = jnp.dot(a, w, preferred_element_type=jnp.float32)


_readout_call = pl.pallas_call(
    _readout_body,
    grid=(1,),
    in_specs=[
        pl.BlockSpec((N_NODES, EMB), lambda i: (0, 0)),
        pl.BlockSpec((ACC_ROWS, EMB), lambda i: (0, 0)),
        pl.BlockSpec((1, N_NODES), lambda i: (0, 0)),
        pl.BlockSpec((1, N_NODES), lambda i: (0, 0)),
        pl.BlockSpec((1, N_NODES), lambda i: (0, 0)),
    ],
    out_specs=pl.BlockSpec((N_GRAPHS, EMB), lambda i: (0, 0)),
    out_shape=jax.ShapeDtypeStruct((N_GRAPHS, EMB), jnp.float32),
)


def kernel(node_feature, qs, W_rel, W_self, edge_index, edge_type,
           node_type, graph_ids):
    src = edge_index[0].astype(jnp.int32)
    dst = edge_index[1].astype(jnp.int32)
    et = edge_type.astype(jnp.int32)
    pad = E_PAD - src.shape[0]
    src_p = jnp.concatenate(
        [src, jnp.zeros((pad,), jnp.int32)]).reshape(N_EDGE_ROWS, EDGE_BLK)
    et_p = jnp.concatenate(
        [et, jnp.zeros((pad,), jnp.int32)]).reshape(N_EDGE_ROWS, EDGE_BLK)
    dst_p = jnp.concatenate(
        [dst, jnp.full((pad,), N_NODES, jnp.int32)]
    ).reshape(N_EDGE_ROWS, EDGE_BLK)
    w_cat = jnp.concatenate([W_rel[0], W_rel[1], W_rel[2], W_self], axis=1)

    y, t2, gidx = _mm_call(node_feature, w_cat, src_p, et_p)
    zeros = jnp.zeros((ACC_ROWS, D_FEAT), jnp.float32)
    acc = _edge_agg_call()(t2, gidx, dst_p, zeros)
    agg = _reduce_call(acc)
    h0 = lax.slice(y, (0, D_FEAT - EMB), (N_NODES, D_FEAT))
    q_tot = _readout_call(
        h0, agg,
        qs.reshape(1, N_NODES),
        node_type.astype(jnp.int32).reshape(1, N_NODES),
        graph_ids.astype(jnp.int32).reshape(1, N_NODES),
    )
    return q_tot


# speedup vs baseline: 15.3098x; 2.5793x over previous
import functools

import jax
import jax.numpy as jnp
from jax import lax
from jax.experimental import pallas as pl
from jax.experimental.pallas import tpu as pltpu
from jax.experimental.pallas import tpu_sc as plsc

N_NODES = 10000
D_FEAT = 128
EMB = 32
N_REL = 3
N_GRAPHS = 64

NW = 32
EDGE_BLK = 128
E_PAD = 327680
N_EDGE_ROWS = E_PAD // EDGE_BLK
BLKS_PER_W = N_EDGE_ROWS // NW
ACC_ROWS = 10112
ROWS_PER_TILE = ACC_ROWS // 16
MM_GRID = 10
MM_BLK = N_NODES // MM_GRID
EROW_BLK = N_EDGE_ROWS // MM_GRID
RED_GRID = 8
RED_BLK = ACC_ROWS // RED_GRID


def _mm_body(x_ref, w_ref, s_ref, t_ref, h0_ref, t2_ref, g_ref):
    r = pl.program_id(1)
    y = jnp.dot(x_ref[...], w_ref[...], preferred_element_type=jnp.float32)
    lane = lax.broadcasted_iota(jnp.int32, (MM_BLK, D_FEAT), 1)
    t2_ref[...] = jnp.where((lane // EMB) == r, y, 0.0)

    @pl.when(r == 0)
    def _():
        h0_ref[...] = lax.slice(y, (0, D_FEAT - EMB), (MM_BLK, D_FEAT))
        g_ref[...] = t_ref[...] * N_NODES + s_ref[...]


_mm_call = pl.pallas_call(
    _mm_body,
    grid=(MM_GRID, N_REL),
    in_specs=[
        pl.BlockSpec((MM_BLK, D_FEAT), lambda i, r: (i, 0)),
        pl.BlockSpec((D_FEAT, D_FEAT), lambda i, r: (0, 0)),
        pl.BlockSpec((EROW_BLK, EDGE_BLK), lambda i, r: (i, 0)),
        pl.BlockSpec((EROW_BLK, EDGE_BLK), lambda i, r: (i, 0)),
    ],
    out_specs=[
        pl.BlockSpec((MM_BLK, EMB), lambda i, r: (i, 0)),
        pl.BlockSpec((MM_BLK, D_FEAT), lambda i, r: (r * MM_GRID + i, 0)),
        pl.BlockSpec((EROW_BLK, EDGE_BLK), lambda i, r: (i, 0)),
    ],
    out_shape=[
        jax.ShapeDtypeStruct((N_NODES, EMB), jnp.float32),
        jax.ShapeDtypeStruct((N_REL * N_NODES, D_FEAT), jnp.float32),
        jax.ShapeDtypeStruct((N_EDGE_ROWS, EDGE_BLK), jnp.int32),
    ],
)


@functools.cache
def _edge_agg_call():
    return functools.partial(
        pl.kernel,
        out_type=jax.ShapeDtypeStruct((2, ACC_ROWS, D_FEAT), jnp.float32),
        mesh=plsc.VectorSubcoreMesh(core_axis_name="c", subcore_axis_name="s",
                                    num_cores=2, num_subcores=16),
        scratch_types=[
            pltpu.VMEM((BLKS_PER_W // 2, EDGE_BLK), jnp.int32),
            pltpu.VMEM((BLKS_PER_W // 2, EDGE_BLK), jnp.int32),
            pltpu.VMEM((EDGE_BLK, D_FEAT), jnp.float32),
            pltpu.VMEM((EDGE_BLK, D_FEAT), jnp.float32),
            pltpu.VMEM_SHARED((ACC_ROWS, D_FEAT), jnp.float32),
            pltpu.SemaphoreType.DMA,
            pltpu.SemaphoreType.DMA,
        ],
    )(_edge_agg)


def _edge_agg(t2_hbm, gidx_hbm, dst_hbm, zeros_hbm, out_hbm,
              gidx_v, dst_v, rows_a, rows_b, acc_sh, sem_a, sem_b):
    cid = lax.axis_index("c")
    sid = lax.axis_index("s")
    wid = cid * 16 + sid
    pltpu.sync_copy(zeros_hbm.at[pl.ds(sid * ROWS_PER_TILE, ROWS_PER_TILE)],
                    acc_sh.at[pl.ds(sid * ROWS_PER_TILE, ROWS_PER_TILE)])
    plsc.subcore_barrier()

    def body(i, carry):
        j = i * 2
        cpa = pltpu.async_copy(t2_hbm.at[gidx_v.at[j]], rows_a, sem_a)
        cpb = pltpu.async_copy(t2_hbm.at[gidx_v.at[j + 1]], rows_b, sem_b)
        cpa.wait()
        pltpu.sync_copy(rows_a, acc_sh.at[dst_v.at[j]], add=True)
        cpb.wait()
        pltpu.sync_copy(rows_b, acc_sh.at[dst_v.at[j + 1]], add=True)
        return carry

    half = BLKS_PER_W // 2
    for p in range(2):
        blk0 = wid * BLKS_PER_W + p * half
        pltpu.sync_copy(gidx_hbm.at[pl.ds(blk0, half)], gidx_v)
        pltpu.sync_copy(dst_hbm.at[pl.ds(blk0, half)], dst_v)
        lax.fori_loop(0, half // 2, body, 0)
    plsc.subcore_barrier()
    pltpu.sync_copy(acc_sh.at[pl.ds(sid * ROWS_PER_TILE, ROWS_PER_TILE)],
                    out_hbm.at[cid, pl.ds(sid * ROWS_PER_TILE, ROWS_PER_TILE)])


def _reduce_body(a_ref, o_ref):
    s = a_ref[0] + a_ref[1]
    o_ref[...] = (s[:, 0:EMB] + s[:, EMB:2 * EMB] + s[:, 2 * EMB:3 * EMB])


_reduce_call = pl.pallas_call(
    _reduce_body,
    grid=(RED_GRID,),
    in_specs=[pl.BlockSpec((2, RED_BLK, D_FEAT), lambda i: (0, i, 0))],
    out_specs=pl.BlockSpec((RED_BLK, EMB), lambda i: (i, 0)),
    out_shape=jax.ShapeDtypeStruct((ACC_ROWS, EMB), jnp.float32),
)


def _readout_body(h0_ref, agg_ref, qs_ref, nt_ref, gid_ref, o_ref):
    h = h0_ref[...] + agg_ref[:N_NODES, :]
    w = jnp.maximum(h, 0.0)
    coef = (qs_ref[...] + 1.0) * (nt_ref[...] == 0).astype(jnp.float32)
    gid = gid_ref[...]
    iota = lax.broadcasted_iota(jnp.int32, (N_GRAPHS, N_NODES), 0)
    a = jnp.where(gid == iota, coef, 0.0)
    o_ref[...] = jnp.dot(a, w, preferred_element_type=jnp.float32)


_readout_call = pl.pallas_call(
    _readout_body,
    grid=(1,),
    in_specs=[
        pl.BlockSpec((N_NODES, EMB), lambda i: (0, 0)),
        pl.BlockSpec((ACC_ROWS, EMB), lambda i: (0, 0)),
        pl.BlockSpec((1, N_NODES), lambda i: (0, 0)),
        pl.BlockSpec((1, N_NODES), lambda i: (0, 0)),
        pl.BlockSpec((1, N_NODES), lambda i: (0, 0)),
    ],
    out_specs=pl.BlockSpec((N_GRAPHS, EMB), lambda i: (0, 0)),
    out_shape=jax.ShapeDtypeStruct((N_GRAPHS, EMB), jnp.float32),
)


def kernel(node_feature, qs, W_rel, W_self, edge_index, edge_type,
           node_type, graph_ids):
    src = edge_index[0].astype(jnp.int32)
    dst = edge_index[1].astype(jnp.int32)
    et = edge_type.astype(jnp.int32)
    pad = E_PAD - src.shape[0]
    ar = jnp.arange(pad, dtype=jnp.int32)
    src_p = jnp.concatenate(
        [src, ar % N_NODES]).reshape(N_EDGE_ROWS, EDGE_BLK)
    et_p = jnp.concatenate(
        [et, jnp.zeros((pad,), jnp.int32)]).reshape(N_EDGE_ROWS, EDGE_BLK)
    dst_p = jnp.concatenate(
        [dst, N_NODES + ar % (ACC_ROWS - N_NODES)]
    ).reshape(N_EDGE_ROWS, EDGE_BLK)
    w_cat = jnp.concatenate([W_rel[0], W_rel[1], W_rel[2], W_self], axis=1)

    h0, t2, gidx = _mm_call(node_feature, w_cat, src_p, et_p)
    zeros = jnp.zeros((ACC_ROWS, D_FEAT), jnp.float32)
    acc = _edge_agg_call()(t2, gidx, dst_p, zeros)
    agg = _reduce_call(acc)
    q_tot = _readout_call(
        h0, agg,
        qs.reshape(1, N_NODES),
        node_type.astype(jnp.int32).reshape(1, N_NODES),
        graph_ids.astype(jnp.int32).reshape(1, N_NODES),
    )
    return q_tot


# speedup vs baseline: 18.6454x; 1.2179x over previous
import functools

import jax
import jax.numpy as jnp
from jax import lax
from jax.experimental import pallas as pl
from jax.experimental.pallas import tpu as pltpu
from jax.experimental.pallas import tpu_sc as plsc

N_NODES = 10000
D_FEAT = 128
EMB = 32
N_REL = 3
N_GRAPHS = 64

NW = 32
EDGE_BLK = 128
E_PAD = 327680
N_EDGE_ROWS = E_PAD // EDGE_BLK
BLKS_PER_W = N_EDGE_ROWS // NW
ACC_ROWS = 10112
ROWS_PER_TILE = ACC_ROWS // 16
MM_GRID = 10
MM_BLK = N_NODES // MM_GRID
EROW_BLK = N_EDGE_ROWS // MM_GRID
RED_GRID = 8
RED_BLK = ACC_ROWS // RED_GRID


def _mm_body(x_ref, w_ref, s_ref, t_ref, h0_ref, t2_ref, g_ref):
    r = pl.program_id(1)
    y = jnp.dot(x_ref[...], w_ref[...], preferred_element_type=jnp.float32)
    lane = lax.broadcasted_iota(jnp.int32, (MM_BLK, D_FEAT), 1)
    t2_ref[...] = jnp.where((lane // EMB) == r, y, 0.0)

    @pl.when(r == 0)
    def _():
        h0_ref[...] = lax.slice(y, (0, D_FEAT - EMB), (MM_BLK, D_FEAT))
        g_ref[...] = t_ref[...] * N_NODES + s_ref[...]


_mm_call = pl.pallas_call(
    _mm_body,
    grid=(MM_GRID, N_REL),
    in_specs=[
        pl.BlockSpec((MM_BLK, D_FEAT), lambda i, r: (i, 0)),
        pl.BlockSpec((D_FEAT, D_FEAT), lambda i, r: (0, 0)),
        pl.BlockSpec((EROW_BLK, EDGE_BLK), lambda i, r: (i, 0)),
        pl.BlockSpec((EROW_BLK, EDGE_BLK), lambda i, r: (i, 0)),
    ],
    out_specs=[
        pl.BlockSpec((MM_BLK, EMB), lambda i, r: (i, 0)),
        pl.BlockSpec((MM_BLK, D_FEAT), lambda i, r: (r * MM_GRID + i, 0)),
        pl.BlockSpec((EROW_BLK, EDGE_BLK), lambda i, r: (i, 0)),
    ],
    out_shape=[
        jax.ShapeDtypeStruct((N_NODES, EMB), jnp.float32),
        jax.ShapeDtypeStruct((N_REL * N_NODES, D_FEAT), jnp.float32),
        jax.ShapeDtypeStruct((N_EDGE_ROWS, EDGE_BLK), jnp.int32),
    ],
)


@functools.cache
def _edge_agg_call():
    return functools.partial(
        pl.kernel,
        out_type=jax.ShapeDtypeStruct((2, ACC_ROWS, D_FEAT), jnp.float32),
        mesh=plsc.VectorSubcoreMesh(core_axis_name="c", subcore_axis_name="s",
                                    num_cores=2, num_subcores=16),
        scratch_types=[
            pltpu.VMEM((BLKS_PER_W // 2, EDGE_BLK), jnp.int32),
            pltpu.VMEM((BLKS_PER_W // 2, EDGE_BLK), jnp.int32),
            pltpu.VMEM((EDGE_BLK, D_FEAT), jnp.float32),
            pltpu.VMEM((EDGE_BLK, D_FEAT), jnp.float32),
            pltpu.VMEM_SHARED((ACC_ROWS, D_FEAT), jnp.float32),
            pltpu.SemaphoreType.DMA,
            pltpu.SemaphoreType.DMA,
            pltpu.SemaphoreType.DMA,
            pltpu.SemaphoreType.DMA,
        ],
    )(_edge_agg)


def _edge_agg(t2_hbm, gidx_hbm, dst_hbm, zeros_hbm, out_hbm,
              gidx_v, dst_v, rows_a, rows_b, acc_sh,
              sem_ga, sem_gb, sem_sa, sem_sb):
    cid = lax.axis_index("c")
    sid = lax.axis_index("s")
    wid = cid * 16 + sid
    half = BLKS_PER_W // 2

    def gather(j, rows, sem):
        return pltpu.async_copy(t2_hbm.at[gidx_v.at[j]], rows, sem)

    def scatter(j, rows, sem):
        return pltpu.async_copy(rows, acc_sh.at[dst_v.at[j]], sem, add=True)

    blk0 = wid * BLKS_PER_W
    pltpu.sync_copy(gidx_hbm.at[pl.ds(blk0, half)], gidx_v)
    pltpu.sync_copy(dst_hbm.at[pl.ds(blk0, half)], dst_v)
    gather(0, rows_a, sem_ga)
    gather(1, rows_b, sem_gb)
    pltpu.sync_copy(zeros_hbm.at[pl.ds(sid * ROWS_PER_TILE, ROWS_PER_TILE)],
                    acc_sh.at[pl.ds(sid * ROWS_PER_TILE, ROWS_PER_TILE)])
    plsc.subcore_barrier()

    def body(i, carry):
        j = i * 2
        pltpu.make_async_copy(t2_hbm.at[gidx_v.at[j]], rows_a, sem_ga).wait()
        scatter(j, rows_a, sem_sa)
        pltpu.make_async_copy(t2_hbm.at[gidx_v.at[j + 1]], rows_b,
                              sem_gb).wait()
        scatter(j + 1, rows_b, sem_sb)

        @pl.when(j + 2 < half)
        def _():
            pltpu.make_async_copy(rows_a, acc_sh.at[dst_v.at[j]],
                                  sem_sa).wait()
            gather(j + 2, rows_a, sem_ga)

        @pl.when(j + 3 < half)
        def _():
            pltpu.make_async_copy(rows_b, acc_sh.at[dst_v.at[j + 1]],
                                  sem_sb).wait()
            gather(j + 3, rows_b, sem_gb)

        return carry

    def drain_and_reload(p):
        pltpu.make_async_copy(rows_a, acc_sh.at[dst_v.at[0]], sem_sa).wait()
        pltpu.make_async_copy(rows_b, acc_sh.at[dst_v.at[1]], sem_sb).wait()
        if p is not None:
            nblk0 = wid * BLKS_PER_W + p * half
            pltpu.sync_copy(gidx_hbm.at[pl.ds(nblk0, half)], gidx_v)
            pltpu.sync_copy(dst_hbm.at[pl.ds(nblk0, half)], dst_v)
            gather(0, rows_a, sem_ga)
            gather(1, rows_b, sem_gb)

    lax.fori_loop(0, half // 2, body, 0)
    drain_and_reload(1)
    lax.fori_loop(0, half // 2, body, 0)
    drain_and_reload(None)
    plsc.subcore_barrier()
    pltpu.sync_copy(acc_sh.at[pl.ds(sid * ROWS_PER_TILE, ROWS_PER_TILE)],
                    out_hbm.at[cid, pl.ds(sid * ROWS_PER_TILE, ROWS_PER_TILE)])


def _reduce_body(a_ref, o_ref):
    s = a_ref[0] + a_ref[1]
    o_ref[...] = (s[:, 0:EMB] + s[:, EMB:2 * EMB] + s[:, 2 * EMB:3 * EMB])


_reduce_call = pl.pallas_call(
    _reduce_body,
    grid=(RED_GRID,),
    in_specs=[pl.BlockSpec((2, RED_BLK, D_FEAT), lambda i: (0, i, 0))],
    out_specs=pl.BlockSpec((RED_BLK, EMB), lambda i: (i, 0)),
    out_shape=jax.ShapeDtypeStruct((ACC_ROWS, EMB), jnp.float32),
)


def _readout_body(h0_ref, agg_ref, qs_ref, nt_ref, gid_ref, o_ref):
    h = h0_ref[...] + agg_ref[:N_NODES, :]
    w = jnp.maximum(h, 0.0)
    coef = (qs_ref[...] + 1.0) * (nt_ref[...] == 0).astype(jnp.float32)
    gid = gid_ref[...]
    iota = lax.broadcasted_iota(jnp.int32, (N_GRAPHS, N_NODES), 0)
    a = jnp.where(gid == iota, coef, 0.0)
    o_ref[...] = jnp.dot(a, w, preferred_element_type=jnp.float32)


_readout_call = pl.pallas_call(
    _readout_body,
    grid=(1,),
    in_specs=[
        pl.BlockSpec((N_NODES, EMB), lambda i: (0, 0)),
        pl.BlockSpec((ACC_ROWS, EMB), lambda i: (0, 0)),
        pl.BlockSpec((1, N_NODES), lambda i: (0, 0)),
        pl.BlockSpec((1, N_NODES), lambda i: (0, 0)),
        pl.BlockSpec((1, N_NODES), lambda i: (0, 0)),
    ],
    out_specs=pl.BlockSpec((N_GRAPHS, EMB), lambda i: (0, 0)),
    out_shape=jax.ShapeDtypeStruct((N_GRAPHS, EMB), jnp.float32),
)


def kernel(node_feature, qs, W_rel, W_self, edge_index, edge_type,
           node_type, graph_ids):
    src = edge_index[0].astype(jnp.int32)
    dst = edge_index[1].astype(jnp.int32)
    et = edge_type.astype(jnp.int32)
    pad = E_PAD - src.shape[0]
    ar = jnp.arange(pad, dtype=jnp.int32)
    src_p = jnp.concatenate(
        [src, ar % N_NODES]).reshape(N_EDGE_ROWS, EDGE_BLK)
    et_p = jnp.concatenate(
        [et, jnp.zeros((pad,), jnp.int32)]).reshape(N_EDGE_ROWS, EDGE_BLK)
    dst_p = jnp.concatenate(
        [dst, N_NODES + ar % (ACC_ROWS - N_NODES)]
    ).reshape(N_EDGE_ROWS, EDGE_BLK)
    w_cat = jnp.concatenate([W_rel[0], W_rel[1], W_rel[2], W_self], axis=1)

    h0, t2, gidx = _mm_call(node_feature, w_cat, src_p, et_p)
    zeros = jnp.zeros((ACC_ROWS, D_FEAT), jnp.float32)
    acc = _edge_agg_call()(t2, gidx, dst_p, zeros)
    agg = _reduce_call(acc)
    q_tot = _readout_call(
        h0, agg,
        qs.reshape(1, N_NODES),
        node_type.astype(jnp.int32).reshape(1, N_NODES),
        graph_ids.astype(jnp.int32).reshape(1, N_NODES),
    )
    return q_tot
